# Initial kernel scaffold; baseline (speedup 1.0000x reference)
import jax
import jax.numpy as jnp
from jax.experimental import pallas as pl


def kernel(inputs_features, features, indices, camids_batch, camids):
    raise NotImplementedError("write your pallas kernel here")


# speedup vs baseline: 1.5383x; 1.5383x over previous
import functools

import jax
import jax.numpy as jnp
from jax.experimental import pallas as pl
from jax.experimental.pallas import tpu as pltpu

B = 64
D = 128
M_TOTAL = 100000
N_CAMS = 6
INV_TEMP = 1.0 / 0.07
BLOCK_M = 8192
NEG = -1e30


def _loss_kernel(x_ref, f_ref, cams_ref, camb_ref, idx_ref, out_ref,
                 m_ref, s_ref, t_ref, *, num_blocks):
    j = pl.program_id(0)

    @pl.when(j == 0)
    def _init():
        m_ref[...] = jnp.full((B, 1), NEG, jnp.float32)
        s_ref[...] = jnp.zeros((B, 1), jnp.float32)
        t_ref[...] = jnp.zeros((B, 1), jnp.float32)

    x = x_ref[...] * INV_TEMP
    f = f_ref[...]
    logits = jax.lax.dot_general(
        x, f, (((1,), (1,)), ((), ())),
        preferred_element_type=jnp.float32)

    cols = j * BLOCK_M + jax.lax.broadcasted_iota(
        jnp.int32, (1, BLOCK_M), 1)
    mask = (camb_ref[...] == cams_ref[...]) & (cols < M_TOTAL)
    ml = jnp.where(mask, logits, NEG)

    m_old = m_ref[...]
    m_new = jnp.maximum(m_old, jnp.max(ml, axis=1, keepdims=True))
    p = jnp.exp(ml - m_new) * mask.astype(jnp.float32)
    s_ref[...] = s_ref[...] * jnp.exp(m_old - m_new) + jnp.sum(
        p, axis=1, keepdims=True)
    m_ref[...] = m_new

    t_ref[...] += jnp.sum(
        jnp.where(cols == idx_ref[...], logits, 0.0), axis=1, keepdims=True)

    @pl.when(j == num_blocks - 1)
    def _fin():
        lse = m_ref[...] + jnp.log(s_ref[...])
        out_ref[...] = jnp.sum((lse - t_ref[...]) / B, axis=(0, 1),
                               keepdims=True)


@jax.jit
def kernel(inputs_features, features, indices, camids_batch, camids):
    num_blocks = pl.cdiv(M_TOTAL, BLOCK_M)
    camids2 = camids.reshape(1, M_TOTAL)
    camb2 = camids_batch.reshape(B, 1)
    idx2 = indices.reshape(B, 1)

    out = pl.pallas_call(
        functools.partial(_loss_kernel, num_blocks=num_blocks),
        grid=(num_blocks,),
        in_specs=[
            pl.BlockSpec((B, D), lambda j: (0, 0)),
            pl.BlockSpec((BLOCK_M, D), lambda j: (j, 0)),
            pl.BlockSpec((1, BLOCK_M), lambda j: (0, j)),
            pl.BlockSpec((B, 1), lambda j: (0, 0)),
            pl.BlockSpec((B, 1), lambda j: (0, 0)),
        ],
        out_specs=pl.BlockSpec((1, 1), lambda j: (0, 0)),
        out_shape=jax.ShapeDtypeStruct((1, 1), jnp.float32),
        scratch_shapes=[
            pltpu.VMEM((B, 1), jnp.float32),
            pltpu.VMEM((B, 1), jnp.float32),
            pltpu.VMEM((B, 1), jnp.float32),
        ],
        compiler_params=pltpu.CompilerParams(
            dimension_semantics=("arbitrary",)),
    )(inputs_features, features, camids2, camb2, idx2)
    return out[0, 0]


# speedup vs baseline: 1.8143x; 1.1795x over previous
import functools

import jax
import jax.numpy as jnp
from jax.experimental import pallas as pl
from jax.experimental.pallas import tpu as pltpu

B = 64
D = 128
M_TOTAL = 100000
INV_TEMP = 1.0 / 0.07
BLOCK_M = 8192
NEG = -1e30


def _loss_kernel(x_ref, f_ref, fany_ref, cams_ref, camb_ref, idx_ref,
                 out_ref, g_ref, m_ref, s_ref, sem, *, num_blocks):
    j = pl.program_id(0)

    @pl.when(j == 0)
    def _init():
        m_ref[...] = jnp.full((B, 1), NEG, jnp.float32)
        s_ref[...] = jnp.zeros((B, 1), jnp.float32)
        for i in range(B):
            pltpu.make_async_copy(
                fany_ref.at[pl.ds(idx_ref[i], 1), :],
                g_ref.at[pl.ds(i, 1), :], sem).start()

    x = x_ref[...] * INV_TEMP
    f = f_ref[...]
    logits = jax.lax.dot_general(
        x, f, (((1,), (1,)), ((), ())),
        preferred_element_type=jnp.float32)

    cols = j * BLOCK_M + jax.lax.broadcasted_iota(
        jnp.int32, (1, BLOCK_M), 1)
    cams = jnp.where(cols < M_TOTAL, cams_ref[...], -1)
    ml = jnp.where(camb_ref[...] == cams, logits, NEG)

    m_old = m_ref[...]
    m_new = jnp.maximum(m_old, jnp.max(ml, axis=1, keepdims=True))
    s_ref[...] = s_ref[...] * jnp.exp(m_old - m_new) + jnp.sum(
        jnp.exp(ml - m_new), axis=1, keepdims=True)
    m_ref[...] = m_new

    @pl.when(j == num_blocks - 1)
    def _fin():
        for i in range(B):
            pltpu.make_async_copy(
                fany_ref.at[pl.ds(idx_ref[i], 1), :],
                g_ref.at[pl.ds(i, 1), :], sem).wait()
        t = jnp.sum(x * g_ref[...], axis=1, keepdims=True)
        lse = m_ref[...] + jnp.log(s_ref[...])
        out_ref[...] = jnp.sum((lse - t) / B, axis=(0, 1), keepdims=True)


@jax.jit
def kernel(inputs_features, features, indices, camids_batch, camids):
    num_blocks = pl.cdiv(M_TOTAL, BLOCK_M)
    camids2 = camids.reshape(1, M_TOTAL)
    camb2 = camids_batch.reshape(B, 1)

    out = pl.pallas_call(
        functools.partial(_loss_kernel, num_blocks=num_blocks),
        grid=(num_blocks,),
        in_specs=[
            pl.BlockSpec((B, D), lambda j: (0, 0)),
            pl.BlockSpec((BLOCK_M, D), lambda j: (j, 0)),
            pl.BlockSpec(memory_space=pl.ANY),
            pl.BlockSpec((1, BLOCK_M), lambda j: (0, j)),
            pl.BlockSpec((B, 1), lambda j: (0, 0)),
            pl.BlockSpec(memory_space=pltpu.SMEM),
        ],
        out_specs=pl.BlockSpec((1, 1), lambda j: (0, 0)),
        out_shape=jax.ShapeDtypeStruct((1, 1), jnp.float32),
        scratch_shapes=[
            pltpu.VMEM((B, D), jnp.float32),
            pltpu.VMEM((B, 1), jnp.float32),
            pltpu.VMEM((B, 1), jnp.float32),
            pltpu.SemaphoreType.DMA,
        ],
        compiler_params=pltpu.CompilerParams(
            dimension_semantics=("arbitrary",)),
    )(inputs_features, features, features, camids2, camb2, indices)
    return out[0, 0]


# speedup vs baseline: 2.0516x; 1.1308x over previous
import functools

import jax
import jax.numpy as jnp
from jax.experimental import pallas as pl
from jax.experimental.pallas import tpu as pltpu

B = 64
D = 128
M_TOTAL = 100000
INV_TEMP = 1.0 / 0.07
NCHUNK = 2
CHUNK = 6272
BLOCK_M = NCHUNK * CHUNK
NEG = -1e30


def _loss_kernel(x_ref, f_ref, fany_ref, cams_ref, camb_ref, idx_ref,
                 out_ref, xs_ref, g_ref, m_ref, s_ref, sem, *, num_blocks):
    j = pl.program_id(0)

    @pl.when(j == 0)
    def _init():
        m_ref[...] = jnp.full((B, NCHUNK), NEG, jnp.float32)
        s_ref[...] = jnp.zeros((B, NCHUNK), jnp.float32)
        xs_ref[...] = x_ref[...] * INV_TEMP
        for i in range(B):
            pltpu.make_async_copy(
                fany_ref.at[pl.ds(idx_ref[i], 1), :],
                g_ref.at[pl.ds(i, 1), :], sem).start()

    xs = xs_ref[...]
    camb = camb_ref[...]

    for k in range(NCHUNK):
        f = f_ref[pl.ds(k * CHUNK, CHUNK), :]
        logits = jax.lax.dot_general(
            xs, f, (((1,), (1,)), ((), ())),
            preferred_element_type=jnp.float32)

        cols = j * BLOCK_M + k * CHUNK + jax.lax.broadcasted_iota(
            jnp.int32, (1, CHUNK), 1)
        cams = jnp.where(cols < M_TOTAL,
                         cams_ref[:, pl.ds(k * CHUNK, CHUNK)], -1)
        ml = jnp.where(camb == cams, logits, NEG)

        m_old = m_ref[:, k:k + 1]
        m_new = jnp.maximum(m_old, jnp.max(ml, axis=1, keepdims=True))
        s_ref[:, k:k + 1] = s_ref[:, k:k + 1] * jnp.exp(m_old - m_new) + \
            jnp.sum(jnp.exp(ml - m_new), axis=1, keepdims=True)
        m_ref[:, k:k + 1] = m_new

    @pl.when(j == num_blocks - 1)
    def _fin():
        for i in range(B):
            pltpu.make_async_copy(
                fany_ref.at[pl.ds(idx_ref[i], 1), :],
                g_ref.at[pl.ds(i, 1), :], sem).wait()
        t = jnp.sum(xs * g_ref[...], axis=1, keepdims=True)
        m_all = m_ref[...]
        m_fin = jnp.max(m_all, axis=1, keepdims=True)
        s_fin = jnp.sum(s_ref[...] * jnp.exp(m_all - m_fin),
                        axis=1, keepdims=True)
        lse = m_fin + jnp.log(s_fin)
        out_ref[...] = jnp.sum((lse - t) / B, axis=(0, 1), keepdims=True)


@jax.jit
def kernel(inputs_features, features, indices, camids_batch, camids):
    num_blocks = pl.cdiv(M_TOTAL, BLOCK_M)
    camids2 = camids.reshape(1, M_TOTAL)
    camb2 = camids_batch.reshape(B, 1)

    out = pl.pallas_call(
        functools.partial(_loss_kernel, num_blocks=num_blocks),
        grid=(num_blocks,),
        in_specs=[
            pl.BlockSpec((B, D), lambda j: (0, 0)),
            pl.BlockSpec((BLOCK_M, D), lambda j: (j, 0)),
            pl.BlockSpec(memory_space=pl.ANY),
            pl.BlockSpec((1, BLOCK_M), lambda j: (0, j)),
            pl.BlockSpec((B, 1), lambda j: (0, 0)),
            pl.BlockSpec(memory_space=pltpu.SMEM),
        ],
        out_specs=pl.BlockSpec((1, 1), lambda j: (0, 0)),
        out_shape=jax.ShapeDtypeStruct((1, 1), jnp.float32),
        scratch_shapes=[
            pltpu.VMEM((B, D), jnp.float32),
            pltpu.VMEM((B, D), jnp.float32),
            pltpu.VMEM((B, NCHUNK), jnp.float32),
            pltpu.VMEM((B, NCHUNK), jnp.float32),
            pltpu.SemaphoreType.DMA,
        ],
        compiler_params=pltpu.CompilerParams(
            dimension_semantics=("arbitrary",)),
    )(inputs_features, features, features, camids2, camb2, indices)
    return out[0, 0]


# speedup vs baseline: 2.1760x; 1.0607x over previous
import functools

import jax
import jax.numpy as jnp
from jax.experimental import pallas as pl
from jax.experimental.pallas import tpu as pltpu

B = 64
D = 128
M_TOTAL = 100000
INV_TEMP = 1.0 / 0.07
NSTREAM = 2
NCHUNK = 2
CHUNK = 6272
BLOCK_M = NCHUNK * CHUNK
NUM_BLOCKS = 4
NEG = -1e30


def _loss_kernel(x_ref, f0_ref, f1_ref, fany_ref, cams_ref, camb_ref,
                 idx_ref, out_ref, xs_ref, g_ref, m_ref, s_ref, sem):
    j = pl.program_id(0)

    @pl.when(j == 0)
    def _init():
        m_ref[...] = jnp.full((B, NSTREAM * NCHUNK), NEG, jnp.float32)
        s_ref[...] = jnp.zeros((B, NSTREAM * NCHUNK), jnp.float32)
        xs_ref[...] = x_ref[...] * INV_TEMP
        for i in range(B):
            pltpu.make_async_copy(
                fany_ref.at[pl.ds(idx_ref[i], 1), :],
                g_ref.at[pl.ds(i, 1), :], sem).start()

    xs = xs_ref[...]
    camb = camb_ref[...]

    for p, f_ref in enumerate((f0_ref, f1_ref)):
        for k in range(NCHUNK):
            a = p * NCHUNK + k
            f = f_ref[pl.ds(k * CHUNK, CHUNK), :]
            logits = jax.lax.dot_general(
                xs, f, (((1,), (1,)), ((), ())),
                preferred_element_type=jnp.float32)

            cols = ((NSTREAM * j + p) * BLOCK_M + k * CHUNK
                    + jax.lax.broadcasted_iota(jnp.int32, (1, CHUNK), 1))
            cams = jnp.where(
                cols < M_TOTAL,
                cams_ref[:, pl.ds((p * NCHUNK + k) * CHUNK, CHUNK)], -1)
            ml = jnp.where(camb == cams, logits, NEG)

            m_old = m_ref[:, a:a + 1]
            m_new = jnp.maximum(m_old, jnp.max(ml, axis=1, keepdims=True))
            s_ref[:, a:a + 1] = s_ref[:, a:a + 1] * jnp.exp(m_old - m_new) \
                + jnp.sum(jnp.exp(ml - m_new), axis=1, keepdims=True)
            m_ref[:, a:a + 1] = m_new

    @pl.when(j == NUM_BLOCKS - 1)
    def _fin():
        for i in range(B):
            pltpu.make_async_copy(
                fany_ref.at[pl.ds(idx_ref[i], 1), :],
                g_ref.at[pl.ds(i, 1), :], sem).wait()
        t = jnp.sum(xs * g_ref[...], axis=1, keepdims=True)
        m_all = m_ref[...]
        m_fin = jnp.max(m_all, axis=1, keepdims=True)
        s_fin = jnp.sum(s_ref[...] * jnp.exp(m_all - m_fin),
                        axis=1, keepdims=True)
        lse = m_fin + jnp.log(s_fin)
        out_ref[...] = jnp.sum((lse - t) / B, axis=(0, 1), keepdims=True)


@jax.jit
def kernel(inputs_features, features, indices, camids_batch, camids):
    camids2 = camids.reshape(1, M_TOTAL)
    camb2 = camids_batch.reshape(B, 1)

    out = pl.pallas_call(
        _loss_kernel,
        grid=(NUM_BLOCKS,),
        in_specs=[
            pl.BlockSpec((B, D), lambda j: (0, 0)),
            pl.BlockSpec((BLOCK_M, D), lambda j: (2 * j, 0)),
            pl.BlockSpec((BLOCK_M, D), lambda j: (2 * j + 1, 0)),
            pl.BlockSpec(memory_space=pl.ANY),
            pl.BlockSpec((1, NSTREAM * BLOCK_M), lambda j: (0, j)),
            pl.BlockSpec((B, 1), lambda j: (0, 0)),
            pl.BlockSpec(memory_space=pltpu.SMEM),
        ],
        out_specs=pl.BlockSpec((1, 1), lambda j: (0, 0)),
        out_shape=jax.ShapeDtypeStruct((1, 1), jnp.float32),
        scratch_shapes=[
            pltpu.VMEM((B, D), jnp.float32),
            pltpu.VMEM((B, D), jnp.float32),
            pltpu.VMEM((B, NSTREAM * NCHUNK), jnp.float32),
            pltpu.VMEM((B, NSTREAM * NCHUNK), jnp.float32),
            pltpu.SemaphoreType.DMA,
        ],
        compiler_params=pltpu.CompilerParams(
            dimension_semantics=("arbitrary",)),
    )(inputs_features, features, features, features, camids2, camb2,
      indices)
    return out[0, 0]
